# baseline (device time: 30108 ns/iter reference)
import jax
import jax.numpy as jnp
from jax import lax
from jax.experimental import pallas as pl
from jax.experimental.pallas import tpu as pltpu

N_DEV = 4
N_HOPS = N_DEV - 1
SEED = N_HOPS
G = 8


def kernel(x):
    m_per, n = x.shape
    n_grp = m_per // G

    def body(x_ref, out_ref, comm_ref, send_sems, recv_sems):
        my_pos = lax.axis_index("i")
        left = lax.rem(my_pos + N_DEV - 1, N_DEV)
        right = lax.rem(my_pos + 1, N_DEV)

        barrier_sem = pltpu.get_barrier_semaphore()
        for nbr in (left, right):
            pl.semaphore_signal(
                barrier_sem, inc=1,
                device_id=(nbr,), device_id_type=pl.DeviceIdType.MESH,
            )
        pl.semaphore_wait(barrier_sem, 2)

        def hop(h):
            src = SEED if h == 0 else h - 1
            return pltpu.make_async_remote_copy(
                src_ref=comm_ref.at[src],
                dst_ref=comm_ref.at[h],
                send_sem=send_sems.at[h],
                recv_sem=recv_sems.at[h],
                device_id=(right,),
                device_id_type=pl.DeviceIdType.MESH,
            )

        with jax.named_scope("seed"):
            t = x_ref[:, :]
            rows = m_per
            while rows > 1:
                half = rows // 2
                t = t[:half, :] * t[half:rows, :]
                rows = half
            comm_ref[SEED, :, :] = t
            r0 = hop(0)
            r0.start()

        with jax.named_scope("phaseA"):
            rowmod = lax.rem(
                lax.broadcasted_iota(jnp.int32, (m_per, 1), 0), G
            )
            y = x_ref[:, :]
            for shift in (1, 2, 4):
                pad = jnp.ones((shift, n), dtype=y.dtype)
                sh = jnp.concatenate([pad, y[: m_per - shift, :]], axis=0)
                y = jnp.where(rowmod >= shift, y * sh, y)

        with jax.named_scope("wait0"):
            r0.wait_recv()
            r1 = hop(1)
            r1.start()

        with jax.named_scope("phaseB"):
            e = y.reshape(n_grp, G, n)[:, G - 1, :]
            shift = 1
            while shift < n_grp:
                pad = jnp.ones((shift, n), dtype=e.dtype)
                e = e * jnp.concatenate([pad, e[: n_grp - shift, :]], axis=0)
                shift *= 2

        with jax.named_scope("wait1"):
            r1.wait_recv()
            r2 = hop(2)
            r2.start()

        with jax.named_scope("phaseC"):
            e_excl = jnp.concatenate(
                [jnp.ones((1, n), dtype=e.dtype), e[: n_grp - 1, :]], axis=0
            )

        with jax.named_scope("wait2"):
            r2.wait_recv()

        with jax.named_scope("final"):
            prefix = jnp.ones((1, n), dtype=x_ref.dtype)
            for h in range(N_HOPS):
                v = comm_ref[h, :, :]
                prefix = prefix * jnp.where(h < my_pos, v, jnp.ones_like(v))

            s = e_excl * prefix
            out3 = y.reshape(n_grp, G, n) * s[:, None, :]
            out_ref[:, :] = out3.reshape(m_per, n)

            r0.wait_send()
            r1.wait_send()
            r2.wait_send()

    return pl.pallas_call(
        body,
        out_shape=jax.ShapeDtypeStruct((m_per, n), x.dtype),
        in_specs=[pl.BlockSpec(memory_space=pltpu.VMEM)],
        out_specs=pl.BlockSpec(memory_space=pltpu.VMEM),
        scratch_shapes=[
            pltpu.VMEM((N_HOPS + 1, 1, n), x.dtype),
            pltpu.SemaphoreType.DMA((N_HOPS,)),
            pltpu.SemaphoreType.DMA((N_HOPS,)),
        ],
        compiler_params=pltpu.CompilerParams(collective_id=0),
    )(x)


# device time: 20878 ns/iter; 1.4421x vs baseline; 1.4421x over previous
import jax
import jax.numpy as jnp
from jax import lax
from jax.experimental import pallas as pl
from jax.experimental.pallas import tpu as pltpu

N_DEV = 4
N_HOPS = N_DEV - 1
SEED = N_HOPS


def kernel(x):
    m_per, n = x.shape
    half = m_per // 2

    def body(x_ref, out_ref, xv, comm_ref, in_sems, out_sems,
             send_sems, recv_sems):
        my_pos = lax.axis_index("i")
        left = lax.rem(my_pos + N_DEV - 1, N_DEV)
        right = lax.rem(my_pos + 1, N_DEV)

        cin = [
            pltpu.make_async_copy(
                x_ref.at[pl.ds(i * half, half)],
                xv.at[pl.ds(i * half, half)],
                in_sems.at[i],
            )
            for i in range(2)
        ]
        cin[0].start()
        cin[1].start()

        barrier_sem = pltpu.get_barrier_semaphore()
        for nbr in (left, right):
            pl.semaphore_signal(
                barrier_sem, inc=1,
                device_id=(nbr,), device_id_type=pl.DeviceIdType.MESH,
            )
        pl.semaphore_wait(barrier_sem, 2)

        def hop(h):
            src = SEED if h == 0 else h - 1
            return pltpu.make_async_remote_copy(
                src_ref=comm_ref.at[src],
                dst_ref=comm_ref.at[h],
                send_sem=send_sems.at[h],
                recv_sem=recv_sems.at[h],
                device_id=(right,),
                device_id_type=pl.DeviceIdType.MESH,
            )

        with jax.named_scope("seed"):
            cin[0].wait()
            ta = xv[pl.ds(0, half), :]
            rows = half
            while rows > 1:
                r2 = rows // 2
                ta = ta[:r2, :] * ta[r2:rows, :]
                rows = r2
            cin[1].wait()
            tb = xv[pl.ds(half, half), :]
            rows = half
            while rows > 1:
                r2 = rows // 2
                tb = tb[:r2, :] * tb[r2:rows, :]
                rows = r2
            comm_ref[SEED, :, :] = ta * tb
            r0 = hop(0)
            r0.start()

        with jax.named_scope("phaseA"):
            y = xv[:, :]
            for shift in (1, 2, 4, 8, 16, 32):
                pad = jnp.ones((shift, n), dtype=y.dtype)
                y = y * jnp.concatenate([pad, y[: m_per - shift, :]], axis=0)

        with jax.named_scope("wait0"):
            r0.wait_recv()
            r1 = hop(1)
            r1.start()

        with jax.named_scope("phaseB"):
            for shift in (64, 128, 256):
                pad = jnp.ones((shift, n), dtype=y.dtype)
                y = y * jnp.concatenate([pad, y[: m_per - shift, :]], axis=0)

        with jax.named_scope("wait1"):
            r1.wait_recv()
            r2 = hop(2)
            r2.start()

        with jax.named_scope("phaseC"):
            for shift in (512, 1024):
                pad = jnp.ones((shift, n), dtype=y.dtype)
                y = y * jnp.concatenate([pad, y[: m_per - shift, :]], axis=0)

        with jax.named_scope("wait2"):
            r2.wait_recv()

        with jax.named_scope("final"):
            prefix = jnp.ones((1, n), dtype=y.dtype)
            for h in range(N_HOPS):
                v = comm_ref[h, :, :]
                prefix = prefix * jnp.where(h < my_pos, v, jnp.ones_like(v))

            couts = []
            for i in range(2):
                xv[pl.ds(i * half, half), :] = (
                    y[i * half : (i + 1) * half, :] * prefix
                )
                cp = pltpu.make_async_copy(
                    xv.at[pl.ds(i * half, half)],
                    out_ref.at[pl.ds(i * half, half)],
                    out_sems.at[i],
                )
                cp.start()
                couts.append(cp)

            couts[0].wait()
            couts[1].wait()

            r0.wait_send()
            r1.wait_send()
            r2.wait_send()

    return pl.pallas_call(
        body,
        out_shape=jax.ShapeDtypeStruct((m_per, n), x.dtype),
        in_specs=[pl.BlockSpec(memory_space=pl.ANY)],
        out_specs=pl.BlockSpec(memory_space=pl.ANY),
        scratch_shapes=[
            pltpu.VMEM((m_per, n), x.dtype),
            pltpu.VMEM((N_HOPS + 1, 1, n), x.dtype),
            pltpu.SemaphoreType.DMA((2,)),
            pltpu.SemaphoreType.DMA((2,)),
            pltpu.SemaphoreType.DMA((N_HOPS,)),
            pltpu.SemaphoreType.DMA((N_HOPS,)),
        ],
        compiler_params=pltpu.CompilerParams(collective_id=0),
    )(x)
